# baseline (speedup 1.0000x reference)
import jax
import jax.numpy as jnp
from jax import lax
from jax.experimental import pallas as pl
from jax.experimental.pallas import tpu as pltpu

R = 128
N = 100000
TEMP_EPS = 1e-20

RB = 8
NRB = R // RB


def _onehot_fused_body(l_ref, u_ref, out_ref):
    g = jnp.log(-jnp.log(u_ref[...] + TEMP_EPS) + TEMP_EPS)
    t = l_ref[...] + g
    cols = lax.broadcasted_iota(jnp.int32, t.shape, 1)
    t = jnp.where(cols < N, t, -jnp.inf)
    bmax = jnp.max(t, axis=1, keepdims=True)
    bidx = jnp.min(
        jnp.where(t == bmax, cols, jnp.int32(2**31 - 1)), axis=1, keepdims=True
    )
    out_ref[...] = (cols == bidx).astype(jnp.float32)


_onehot_fused_call = pl.pallas_call(
    _onehot_fused_body,
    out_shape=jax.ShapeDtypeStruct((R, N), jnp.float32),
    grid=(NRB,),
    in_specs=[
        pl.BlockSpec((RB, N), lambda j: (j, 0)),
        pl.BlockSpec((RB, N), lambda j: (j, 0)),
    ],
    out_specs=pl.BlockSpec((RB, N), lambda j: (j, 0)),
    compiler_params=pltpu.CompilerParams(
        dimension_semantics=("parallel",),
    ),
)


def kernel(logits, uniform_noise):
    return _onehot_fused_call(logits, uniform_noise)
